# baseline (speedup 1.0000x reference)
import math

import jax
import jax.numpy as jnp
from jax.experimental import pallas as pl
from jax.experimental.pallas import tpu as pltpu

_LABEL_SMOOTHING = 0.1
_CONFIDENCE = 1.0 - _LABEL_SMOOTHING

_ROWS_PER_BLOCK = 64
_COLS_PER_CHUNK = 2000


def _loss_kernel(fill_term, conf_term, fill, conf, n_chunks,
                 x_ref, lab_ref, o_ref):
    r, v = x_ref.shape
    c = _COLS_PER_CHUNK
    lab = lab_ref[...]

    m = jnp.full((r, 1), -jnp.inf, dtype=jnp.float32)
    for i in range(n_chunks):
        xs = x_ref[:, i * c:(i + 1) * c]
        m = jnp.maximum(m, jnp.max(xs, axis=1, keepdims=True))

    s = jnp.zeros((r, 1), dtype=jnp.float32)
    for i in range(n_chunks):
        xs = x_ref[:, i * c:(i + 1) * c]
        s = s + jnp.sum(jnp.exp(xs - m), axis=1, keepdims=True)

    lse = m + jnp.log(s)
    fill_row = fill_term + fill * lse
    conf_row = conf_term + conf * lse

    for i in range(n_chunks):
        xs = x_ref[:, i * c:(i + 1) * c]
        cols = i * c + jax.lax.broadcasted_iota(jnp.int32, (r, c), 1)
        eq = cols == lab
        coef = jnp.where(eq, conf, fill)
        const = jnp.where(eq, conf_row, fill_row)
        o_ref[:, i * c:(i + 1) * c] = const - coef * xs


def kernel(dec_outs, labels):
    n, v = dec_outs.shape
    fill = _LABEL_SMOOTHING / (v - 1)
    fill_term = fill * math.log(fill)
    conf = _CONFIDENCE
    conf_term = conf * math.log(conf)

    r = _ROWS_PER_BLOCK
    c = _COLS_PER_CHUNK
    assert v % c == 0
    n_chunks = v // c
    grid = (n // r,)
    lab2d = labels.reshape(n, 1)

    return pl.pallas_call(
        lambda x_ref, lab_ref, o_ref: _loss_kernel(
            fill_term, conf_term, fill, conf, n_chunks,
            x_ref, lab_ref, o_ref),
        grid=grid,
        in_specs=[
            pl.BlockSpec((r, v), lambda i: (i, 0)),
            pl.BlockSpec((r, 1), lambda i: (i, 0)),
        ],
        out_specs=pl.BlockSpec((r, v), lambda i: (i, 0)),
        out_shape=jax.ShapeDtypeStruct((n, v), dec_outs.dtype),
        compiler_params=pltpu.CompilerParams(
            dimension_semantics=("parallel",),
        ),
    )(dec_outs, lab2d)


# speedup vs baseline: 1.5151x; 1.5151x over previous
import math

import jax
import jax.numpy as jnp
from jax.experimental import pallas as pl
from jax.experimental.pallas import tpu as pltpu

_LABEL_SMOOTHING = 0.1
_CONFIDENCE = 1.0 - _LABEL_SMOOTHING

_ROWS_PER_BLOCK = 64
_COLS_PER_CHUNK = 640


def _loss_kernel(fill_term, conf_term, fill, conf, n_chunks,
                 x_ref, lab_ref, o_ref):
    r, v = x_ref.shape
    c = _COLS_PER_CHUNK
    lab = lab_ref[...]

    x = x_ref[...]
    m = jnp.max(x, axis=1, keepdims=True)
    s = jnp.sum(jnp.exp(x - m), axis=1, keepdims=True)
    lse = m + jnp.log(s)
    fill_row = fill_term + fill * lse
    conf_row = conf_term + conf * lse

    for i in range(n_chunks):
        xs = x_ref[:, i * c:(i + 1) * c]
        cols = i * c + jax.lax.broadcasted_iota(jnp.int32, (r, c), 1)
        eq = cols == lab
        coef = jnp.where(eq, conf, fill)
        const = jnp.where(eq, conf_row, fill_row)
        o_ref[:, i * c:(i + 1) * c] = const - coef * xs


def kernel(dec_outs, labels):
    n, v = dec_outs.shape
    fill = _LABEL_SMOOTHING / (v - 1)
    fill_term = fill * math.log(fill)
    conf = _CONFIDENCE
    conf_term = conf * math.log(conf)

    r = _ROWS_PER_BLOCK
    c = _COLS_PER_CHUNK
    assert v % c == 0
    n_chunks = v // c
    grid = (n // r,)
    lab2d = labels.reshape(n, 1)

    return pl.pallas_call(
        lambda x_ref, lab_ref, o_ref: _loss_kernel(
            fill_term, conf_term, fill, conf, n_chunks,
            x_ref, lab_ref, o_ref),
        grid=grid,
        in_specs=[
            pl.BlockSpec((r, v), lambda i: (i, 0)),
            pl.BlockSpec((r, 1), lambda i: (i, 0)),
        ],
        out_specs=pl.BlockSpec((r, v), lambda i: (i, 0)),
        out_shape=jax.ShapeDtypeStruct((n, v), dec_outs.dtype),
        compiler_params=pltpu.CompilerParams(
            dimension_semantics=("parallel",),
        ),
    )(dec_outs, lab2d)
